# baseline (speedup 1.0000x reference)
import functools

import jax
import jax.numpy as jnp
from jax.experimental import pallas as pl
from jax.experimental.pallas import tpu as pltpu

EPS = 1e-05


def _ln_body(len_ref, x_ref, w_ref, b_ref, o_ref, *, block_s, feat):
    b = pl.program_id(0)
    sb = pl.program_id(1)
    length = len_ref[b]
    x = x_ref[0]
    rows = sb * block_s + jax.lax.broadcasted_iota(jnp.int32, (block_s, 1), 0)
    mask = rows < length
    xm = jnp.where(mask, x, 0.0)
    mean = jnp.sum(xm, axis=1, keepdims=True) * (1.0 / feat)
    xc = jnp.where(mask, xm - mean, 0.0)
    var = jnp.sum(xc * xc, axis=1, keepdims=True) * (1.0 / feat)
    inv = 1.0 / (jnp.sqrt(var) + EPS)
    y = xc * inv * w_ref[0] + b_ref[0]
    o_ref[0] = jnp.where(mask, y, 0.0)


def kernel(x, weights, biases, lengths):
    B, S, F = x.shape
    block_s = 256
    grid = (B, S // block_s)
    lengths32 = lengths.astype(jnp.int32)
    w2 = weights.reshape(1, F)
    b2 = biases.reshape(1, F)
    return pl.pallas_call(
        functools.partial(_ln_body, block_s=block_s, feat=F),
        grid=grid,
        in_specs=[
            pl.BlockSpec(memory_space=pltpu.SMEM),
            pl.BlockSpec((1, block_s, F), lambda b, s: (b, s, 0)),
            pl.BlockSpec((1, F), lambda b, s: (0, 0)),
            pl.BlockSpec((1, F), lambda b, s: (0, 0)),
        ],
        out_specs=pl.BlockSpec((1, block_s, F), lambda b, s: (b, s, 0)),
        out_shape=jax.ShapeDtypeStruct((B, S, F), x.dtype),
        compiler_params=pltpu.CompilerParams(
            dimension_semantics=("parallel", "parallel"),
        ),
    )(lengths32, x, w2, b2)


# speedup vs baseline: 1.0242x; 1.0242x over previous
import functools

import jax
import jax.numpy as jnp
from jax.experimental import pallas as pl
from jax.experimental.pallas import tpu as pltpu

EPS = 1e-05


def _ln_body(len_ref, x_ref, w_ref, b_ref, o_ref, *, block_s, feat):
    b = pl.program_id(0)
    sb = pl.program_id(1)
    length = len_ref[b]
    rows = sb * block_s + jax.lax.broadcasted_iota(jnp.int32, (block_s, 1), 0)
    mask = rows < length

    @pl.when(sb * block_s < length)
    def _():
        x = x_ref[0]
        mean = jnp.sum(x, axis=1, keepdims=True) * (1.0 / feat)
        xc = x - mean
        var = jnp.sum(xc * xc, axis=1, keepdims=True) * (1.0 / feat)
        inv = 1.0 / (jnp.sqrt(var) + EPS)
        y = xc * inv * w_ref[0] + b_ref[0]
        o_ref[0] = jnp.where(mask, y, 0.0)

    @pl.when(sb * block_s >= length)
    def _():
        o_ref[0] = jnp.zeros((block_s, feat), o_ref.dtype)


def kernel(x, weights, biases, lengths):
    B, S, F = x.shape
    block_s = 256
    grid = (B, S // block_s)
    lengths32 = lengths.astype(jnp.int32)
    w2 = weights.reshape(1, F)
    b2 = biases.reshape(1, F)

    def x_index(b, s, len_ref):
        last = jnp.maximum((len_ref[b] + block_s - 1) // block_s - 1, 0)
        return (b, jnp.minimum(s, last), 0)

    grid_spec = pltpu.PrefetchScalarGridSpec(
        num_scalar_prefetch=1,
        grid=grid,
        in_specs=[
            pl.BlockSpec((1, block_s, F), x_index),
            pl.BlockSpec((1, F), lambda b, s, len_ref: (0, 0)),
            pl.BlockSpec((1, F), lambda b, s, len_ref: (0, 0)),
        ],
        out_specs=pl.BlockSpec((1, block_s, F), lambda b, s, len_ref: (b, s, 0)),
    )
    return pl.pallas_call(
        functools.partial(_ln_body, block_s=block_s, feat=F),
        grid_spec=grid_spec,
        out_shape=jax.ShapeDtypeStruct((B, S, F), x.dtype),
        compiler_params=pltpu.CompilerParams(
            dimension_semantics=("parallel", "arbitrary"),
        ),
    )(lengths32, x, w2, b2)


# speedup vs baseline: 1.3316x; 1.3002x over previous
import functools

import jax
import jax.numpy as jnp
from jax.experimental import pallas as pl
from jax.experimental.pallas import tpu as pltpu

EPS = 1e-05


def _ln_body(len_ref, x_ref, w_ref, b_ref, o_ref, *, block_s, feat):
    b = pl.program_id(0)
    sb = pl.program_id(1)
    length = len_ref[b]
    rows = sb * block_s + jax.lax.broadcasted_iota(jnp.int32, (block_s, 1), 0)
    mask = rows < length

    @pl.when(sb * block_s < length)
    def _():
        x = x_ref[0]
        mean = jnp.sum(x, axis=1, keepdims=True) * (1.0 / feat)
        xc = x - mean
        var = jnp.sum(xc * xc, axis=1, keepdims=True) * (1.0 / feat)
        inv = 1.0 / (jnp.sqrt(var) + EPS)
        y = xc * inv * w_ref[0] + b_ref[0]
        o_ref[0] = jnp.where(mask, y, 0.0)

    @pl.when(sb * block_s >= length)
    def _():
        o_ref[0] = jnp.zeros((block_s, feat), o_ref.dtype)


def kernel(x, weights, biases, lengths):
    B, S, F = x.shape
    block_s = 512
    grid = (B, S // block_s)
    lengths32 = lengths.astype(jnp.int32)
    w2 = weights.reshape(1, F)
    b2 = biases.reshape(1, F)

    def x_index(b, s, len_ref):
        last = jnp.maximum((len_ref[b] + block_s - 1) // block_s - 1, 0)
        return (b, jnp.minimum(s, last), 0)

    grid_spec = pltpu.PrefetchScalarGridSpec(
        num_scalar_prefetch=1,
        grid=grid,
        in_specs=[
            pl.BlockSpec((1, block_s, F), x_index),
            pl.BlockSpec((1, F), lambda b, s, len_ref: (0, 0)),
            pl.BlockSpec((1, F), lambda b, s, len_ref: (0, 0)),
        ],
        out_specs=pl.BlockSpec((1, block_s, F), lambda b, s, len_ref: (b, s, 0)),
    )
    return pl.pallas_call(
        functools.partial(_ln_body, block_s=block_s, feat=F),
        grid_spec=grid_spec,
        out_shape=jax.ShapeDtypeStruct((B, S, F), x.dtype),
        compiler_params=pltpu.CompilerParams(
            dimension_semantics=("parallel", "arbitrary"),
        ),
    )(lengths32, x, w2, b2)


# speedup vs baseline: 1.9060x; 1.4314x over previous
import functools

import jax
import jax.numpy as jnp
from jax.experimental import pallas as pl
from jax.experimental.pallas import tpu as pltpu

EPS = 1e-05


def _ln_body(len_ref, x_ref, w_ref, b_ref, o_ref, *, block_s, feat):
    b = pl.program_id(0)
    sb = pl.program_id(1)
    length = len_ref[b]
    rows = sb * block_s + jax.lax.broadcasted_iota(jnp.int32, (block_s, 1), 0)
    mask = rows < length

    @pl.when(sb * block_s < length)
    def _():
        x = x_ref[0]
        mean = jnp.sum(x, axis=1, keepdims=True) * (1.0 / feat)
        xc = x - mean
        var = jnp.sum(xc * xc, axis=1, keepdims=True) * (1.0 / feat)
        inv = 1.0 / (jnp.sqrt(var) + EPS)
        y = xc * inv * w_ref[0] + b_ref[0]
        o_ref[0] = jnp.where(mask, y, 0.0)

    @pl.when(sb * block_s >= length)
    def _():
        o_ref[0] = jnp.zeros((block_s, feat), o_ref.dtype)


def kernel(x, weights, biases, lengths):
    B, S, F = x.shape
    block_s = 1024
    grid = (B, S // block_s)
    lengths32 = lengths.astype(jnp.int32)
    w2 = weights.reshape(1, F)
    b2 = biases.reshape(1, F)

    def x_index(b, s, len_ref):
        last = jnp.maximum((len_ref[b] + block_s - 1) // block_s - 1, 0)
        return (b, jnp.minimum(s, last), 0)

    grid_spec = pltpu.PrefetchScalarGridSpec(
        num_scalar_prefetch=1,
        grid=grid,
        in_specs=[
            pl.BlockSpec((1, block_s, F), x_index),
            pl.BlockSpec((1, F), lambda b, s, len_ref: (0, 0)),
            pl.BlockSpec((1, F), lambda b, s, len_ref: (0, 0)),
        ],
        out_specs=pl.BlockSpec((1, block_s, F), lambda b, s, len_ref: (b, s, 0)),
    )
    return pl.pallas_call(
        functools.partial(_ln_body, block_s=block_s, feat=F),
        grid_spec=grid_spec,
        out_shape=jax.ShapeDtypeStruct((B, S, F), x.dtype),
        compiler_params=pltpu.CompilerParams(
            dimension_semantics=("parallel", "arbitrary"),
        ),
    )(lengths32, x, w2, b2)


# speedup vs baseline: 2.3109x; 1.2124x over previous
import functools

import jax
import jax.numpy as jnp
from jax.experimental import pallas as pl
from jax.experimental.pallas import tpu as pltpu

EPS = 1e-05


def _ln_body(len_ref, x_ref, w_ref, b_ref, o_ref, *, batches_per_block, seq, feat):
    step = pl.program_id(0)
    w = w_ref[0]
    bias = b_ref[0]
    row_iota = jax.lax.broadcasted_iota(jnp.int32, (seq, 1), 0)
    for i in range(batches_per_block):
        length = len_ref[step * batches_per_block + i]
        mask = row_iota < length
        x = x_ref[i]
        mean = jnp.sum(x, axis=1, keepdims=True) * (1.0 / feat)
        xc = x - mean
        var = jnp.sum(xc * xc, axis=1, keepdims=True) * (1.0 / feat)
        inv = 1.0 / (jnp.sqrt(var) + EPS)
        y = xc * inv * w + bias
        o_ref[i] = jnp.where(mask, y, 0.0)


def kernel(x, weights, biases, lengths):
    B, S, F = x.shape
    bpb = 2
    grid = (B // bpb,)
    lengths32 = lengths.astype(jnp.int32)
    w2 = weights.reshape(1, F)
    b2 = biases.reshape(1, F)

    grid_spec = pltpu.PrefetchScalarGridSpec(
        num_scalar_prefetch=1,
        grid=grid,
        in_specs=[
            pl.BlockSpec((bpb, S, F), lambda i, len_ref: (i, 0, 0)),
            pl.BlockSpec((1, F), lambda i, len_ref: (0, 0)),
            pl.BlockSpec((1, F), lambda i, len_ref: (0, 0)),
        ],
        out_specs=pl.BlockSpec((bpb, S, F), lambda i, len_ref: (i, 0, 0)),
    )
    return pl.pallas_call(
        functools.partial(_ln_body, batches_per_block=bpb, seq=S, feat=F),
        grid_spec=grid_spec,
        out_shape=jax.ShapeDtypeStruct((B, S, F), x.dtype),
        compiler_params=pltpu.CompilerParams(
            dimension_semantics=("parallel",),
        ),
    )(lengths32, x, w2, b2)


# speedup vs baseline: 2.4271x; 1.0503x over previous
import functools

import jax
import jax.numpy as jnp
from jax.experimental import pallas as pl
from jax.experimental.pallas import tpu as pltpu

EPS = 1e-05


def _ln_body(len_ref, x_ref, w_ref, b_ref, o_ref, *, batches_per_block, seq, feat):
    step = pl.program_id(0)
    w = w_ref[0]
    bias = b_ref[0]
    row_iota = jax.lax.broadcasted_iota(jnp.int32, (seq, 1), 0)
    for i in range(batches_per_block):
        length = len_ref[step * batches_per_block + i]
        mask = row_iota < length
        x = x_ref[i]
        mean = jnp.sum(x, axis=1, keepdims=True) * (1.0 / feat)
        xc = x - mean
        var = jnp.sum(xc * xc, axis=1, keepdims=True) * (1.0 / feat)
        inv = 1.0 / (jnp.sqrt(var) + EPS)
        y = xc * inv * w + bias
        o_ref[i] = jnp.where(mask, y, 0.0)


def kernel(x, weights, biases, lengths):
    B, S, F = x.shape
    bpb = 4
    grid = (B // bpb,)
    lengths32 = lengths.astype(jnp.int32)
    w2 = weights.reshape(1, F)
    b2 = biases.reshape(1, F)

    grid_spec = pltpu.PrefetchScalarGridSpec(
        num_scalar_prefetch=1,
        grid=grid,
        in_specs=[
            pl.BlockSpec((bpb, S, F), lambda i, len_ref: (i, 0, 0)),
            pl.BlockSpec((1, F), lambda i, len_ref: (0, 0)),
            pl.BlockSpec((1, F), lambda i, len_ref: (0, 0)),
        ],
        out_specs=pl.BlockSpec((bpb, S, F), lambda i, len_ref: (i, 0, 0)),
    )
    return pl.pallas_call(
        functools.partial(_ln_body, batches_per_block=bpb, seq=S, feat=F),
        grid_spec=grid_spec,
        out_shape=jax.ShapeDtypeStruct((B, S, F), x.dtype),
        compiler_params=pltpu.CompilerParams(
            dimension_semantics=("parallel",),
        ),
    )(lengths32, x, w2, b2)


# speedup vs baseline: 2.9523x; 1.2164x over previous
import functools

import jax
import jax.numpy as jnp
from jax.experimental import pallas as pl
from jax.experimental.pallas import tpu as pltpu

EPS = 1e-05


def _ln_body(len_ref, x_hbm, w_ref, b_ref, o_ref, x_buf, sems, *,
             bpb, seq, feat, ch, steps_per_core):
    core = pl.program_id(0)
    step = pl.program_id(1)
    gstep = core * steps_per_core + step
    chunks = seq // ch

    def chunk_copy(g, slot, i, c):
        b = g * bpb + i
        return pltpu.make_async_copy(
            x_hbm.at[b, pl.ds(c * ch, ch), :],
            x_buf.at[slot, i, pl.ds(c * ch, ch), :],
            sems.at[slot, i, c],
        )

    def start_copies(g, slot):
        for i in range(bpb):
            length = len_ref[g * bpb + i]
            for c in range(chunks):
                @pl.when(c * ch < length)
                def _():
                    chunk_copy(g, slot, i, c).start()

    slot = step % 2

    @pl.when(step == 0)
    def _():
        start_copies(gstep, 0)

    @pl.when(step + 1 < steps_per_core)
    def _():
        start_copies(gstep + 1, (step + 1) % 2)

    w = w_ref[0]
    bias = b_ref[0]
    row_iota = jax.lax.broadcasted_iota(jnp.int32, (seq, 1), 0)
    for i in range(bpb):
        length = len_ref[gstep * bpb + i]
        for c in range(chunks):
            @pl.when(c * ch < length)
            def _():
                chunk_copy(gstep, slot, i, c).wait()
        x = x_buf[slot, i]
        mean = jnp.sum(x, axis=1, keepdims=True) * (1.0 / feat)
        xc = x - mean
        var = jnp.sum(xc * xc, axis=1, keepdims=True) * (1.0 / feat)
        inv = 1.0 / (jnp.sqrt(var) + EPS)
        y = xc * inv * w + bias
        o_ref[i] = jnp.where(row_iota < length, y, 0.0)


def kernel(x, weights, biases, lengths):
    B, S, F = x.shape
    bpb = 4
    ch = 128
    cores = 2
    steps_per_core = B // bpb // cores
    lengths32 = lengths.astype(jnp.int32)
    w2 = weights.reshape(1, F)
    b2 = biases.reshape(1, F)

    grid_spec = pltpu.PrefetchScalarGridSpec(
        num_scalar_prefetch=1,
        grid=(cores, steps_per_core),
        in_specs=[
            pl.BlockSpec(memory_space=pl.ANY),
            pl.BlockSpec((1, F), lambda co, s, len_ref: (0, 0)),
            pl.BlockSpec((1, F), lambda co, s, len_ref: (0, 0)),
        ],
        out_specs=pl.BlockSpec(
            (bpb, S, F),
            lambda co, s, len_ref: (co * (B // bpb // cores) + s, 0, 0),
        ),
        scratch_shapes=[
            pltpu.VMEM((2, bpb, S, F), jnp.float32),
            pltpu.SemaphoreType.DMA((2, bpb, S // ch)),
        ],
    )
    return pl.pallas_call(
        functools.partial(_ln_body, bpb=bpb, seq=S, feat=F, ch=ch,
                          steps_per_core=steps_per_core),
        grid_spec=grid_spec,
        out_shape=jax.ShapeDtypeStruct((B, S, F), x.dtype),
        compiler_params=pltpu.CompilerParams(
            dimension_semantics=("parallel", "arbitrary"),
        ),
    )(lengths32, x, w2, b2)


# speedup vs baseline: 2.9598x; 1.0025x over previous
import functools

import jax
import jax.numpy as jnp
from jax.experimental import pallas as pl
from jax.experimental.pallas import tpu as pltpu

EPS = 1e-05


def _ln_body(len_ref, x_hbm, w_ref, b_ref, o_ref, x_buf, sems, *,
             bpb, seq, feat, ch, steps_per_core):
    core = pl.program_id(0)
    step = pl.program_id(1)
    gstep = core * steps_per_core + step
    chunks = seq // ch

    def chunk_copy(g, slot, i, c):
        b = g * bpb + i
        return pltpu.make_async_copy(
            x_hbm.at[b, pl.ds(c * ch, ch), :],
            x_buf.at[slot, i, pl.ds(c * ch, ch), :],
            sems.at[slot, i, c],
        )

    def start_copies(g, slot):
        for i in range(bpb):
            length = len_ref[g * bpb + i]
            for c in range(chunks):
                @pl.when(c * ch < length)
                def _():
                    chunk_copy(g, slot, i, c).start()

    slot = step % 2

    @pl.when(step == 0)
    def _():
        start_copies(gstep, 0)

    @pl.when(step + 1 < steps_per_core)
    def _():
        start_copies(gstep + 1, (step + 1) % 2)

    w = w_ref[0]
    bias = b_ref[0]
    row_iota = jax.lax.broadcasted_iota(jnp.int32, (seq, 1), 0)
    for i in range(bpb):
        length = len_ref[gstep * bpb + i]
        for c in range(chunks):
            @pl.when(c * ch < length)
            def _():
                chunk_copy(gstep, slot, i, c).wait()
        x = x_buf[slot, i]
        mean = jnp.sum(x, axis=1, keepdims=True) * (1.0 / feat)
        xc = x - mean
        var = jnp.sum(xc * xc, axis=1, keepdims=True) * (1.0 / feat)
        inv = 1.0 / (jnp.sqrt(var) + EPS)
        y = xc * inv * w + bias
        o_ref[i] = jnp.where(row_iota < length, y, 0.0)


def kernel(x, weights, biases, lengths):
    B, S, F = x.shape
    bpb = 4
    ch = 64
    cores = 2
    steps_per_core = B // bpb // cores
    lengths32 = lengths.astype(jnp.int32)
    w2 = weights.reshape(1, F)
    b2 = biases.reshape(1, F)

    grid_spec = pltpu.PrefetchScalarGridSpec(
        num_scalar_prefetch=1,
        grid=(cores, steps_per_core),
        in_specs=[
            pl.BlockSpec(memory_space=pl.ANY),
            pl.BlockSpec((1, F), lambda co, s, len_ref: (0, 0)),
            pl.BlockSpec((1, F), lambda co, s, len_ref: (0, 0)),
        ],
        out_specs=pl.BlockSpec(
            (bpb, S, F),
            lambda co, s, len_ref: (co * (B // bpb // cores) + s, 0, 0),
        ),
        scratch_shapes=[
            pltpu.VMEM((2, bpb, S, F), jnp.float32),
            pltpu.SemaphoreType.DMA((2, bpb, S // ch)),
        ],
    )
    return pl.pallas_call(
        functools.partial(_ln_body, bpb=bpb, seq=S, feat=F, ch=ch,
                          steps_per_core=steps_per_core),
        grid_spec=grid_spec,
        out_shape=jax.ShapeDtypeStruct((B, S, F), x.dtype),
        compiler_params=pltpu.CompilerParams(
            dimension_semantics=("parallel", "arbitrary"),
        ),
    )(lengths32, x, w2, b2)
